# Initial kernel scaffold; baseline (speedup 1.0000x reference)
import jax
import jax.numpy as jnp
from jax.experimental import pallas as pl


def kernel(input_xyz, sa1_xyz, sa2_xyz, input_features, sa1_features, backbone_feat, fp1_w1, fp1_b1, fp1_g1, fp1_be1, fp1_w2, fp1_b2, fp1_g2, fp1_be2, fp2_w1, fp2_b1, fp2_g1, fp2_be1, fp2_w2, fp2_b2, fp2_g2, fp2_be2, cls_g, cls_be, cls_w, cls_b):
    raise NotImplementedError("write your pallas kernel here")


# speedup vs baseline: 24.3468x; 24.3468x over previous
import functools

import jax
import jax.numpy as jnp
from jax.experimental import pallas as pl

B = 8
N = 4096
N1 = 1024
N2 = 256
DIN = 3
NC = 20

_EPS_D = 1e-8
_EPS_BN = 1e-5
_BIG_I = 2**30


def _top3_weights(d, m):
    iota0 = jax.lax.broadcasted_iota(jnp.int32, d.shape, 0)
    dks = []
    idxs = []
    for _ in range(3):
        mval = jnp.min(d, axis=0, keepdims=True)
        hit = d == mval
        idxk = jnp.min(jnp.where(hit, iota0, _BIG_I), axis=0, keepdims=True)
        dks.append(mval)
        idxs.append(idxk)
        d = jnp.where(iota0 == idxk, jnp.inf, d)
    recips = [1.0 / (dk + _EPS_D) for dk in dks]
    norm = recips[0] + recips[1] + recips[2]
    wt = jnp.zeros(d.shape, jnp.float32)
    for rk, idxk in zip(recips, idxs):
        wt = jnp.where(iota0 == idxk, rk / norm, wt)
    return wt


def _sqdist(kxyz, uxyz_c, m, n):
    d = jnp.zeros((m, n), jnp.float32)
    for c in range(3):
        diff = kxyz[:, c:c + 1] - uxyz_c[c:c + 1, :]
        d = d + diff * diff
    return d


def _accum_sums(s_ref, y, first):
    part = jnp.concatenate(
        [jnp.sum(y, axis=1, keepdims=True),
         jnp.sum(y * y, axis=1, keepdims=True)], axis=1)

    @pl.when(first)
    def _():
        s_ref[...] = jnp.zeros_like(s_ref)

    s_ref[...] += part


def _fp1_body(uxyz_ref, kxyz_ref, w1a_ref, bb_ref, f_ref, w1b_ref, b1_ref,
              y_ref, s_ref):
    b = pl.program_id(0)
    d = _sqdist(kxyz_ref[0], uxyz_ref[0], N2, N1)
    wt = _top3_weights(d, N2)
    hfeat = jnp.dot(w1a_ref[...], bb_ref[0], preferred_element_type=jnp.float32)
    y = jnp.dot(hfeat, wt, preferred_element_type=jnp.float32)
    y = y + jnp.dot(w1b_ref[...], f_ref[0], preferred_element_type=jnp.float32)
    y = y + b1_ref[...]
    y_ref[0] = y
    _accum_sums(s_ref, y, b == 0)


def _bn_scale_shift(s_ref, g_ref, be_ref, count):
    s = s_ref[...]
    mean = s[:, 0:1] / count
    var = s[:, 1:2] / count - mean * mean
    inv = jax.lax.rsqrt(var + _EPS_BN)
    scale = g_ref[...] * inv
    shift = be_ref[...] - mean * scale
    return scale, shift


def _bn_relu_conv_body(count, x_ref, s_in_ref, g_ref, be_ref, w_ref, b_ref,
                       z_ref, s_out_ref):
    first = pl.program_id(0) == 0
    scale, shift = _bn_scale_shift(s_in_ref, g_ref, be_ref, count)
    a = jnp.maximum(x_ref[0] * scale + shift, 0.0)
    z = jnp.dot(w_ref[...], a, preferred_element_type=jnp.float32)
    if b_ref is not None:
        z = z + b_ref[...]
    z_ref[0] = z
    if s_out_ref is not None:
        _accum_sums(s_out_ref, z, first)


def _bn_relu_conv_body_nosum(count, x_ref, s_in_ref, g_ref, be_ref, w_ref,
                             b_ref, z_ref):
    _bn_relu_conv_body(count, x_ref, s_in_ref, g_ref, be_ref, w_ref, b_ref,
                       z_ref, None)


def _bn_relu_conv_body_nobias(count, x_ref, s_in_ref, g_ref, be_ref, w_ref,
                              z_ref):
    _bn_relu_conv_body(count, x_ref, s_in_ref, g_ref, be_ref, w_ref, None,
                       z_ref, None)


def _bn_relu_body(count, x_ref, s_in_ref, g_ref, be_ref, f_ref, s_out_ref):
    first = pl.program_id(0) == 0
    scale, shift = _bn_scale_shift(s_in_ref, g_ref, be_ref, count)
    f = jnp.maximum(x_ref[0] * scale + shift, 0.0)
    f_ref[0] = f
    _accum_sums(s_out_ref, f, first)


def _fp2_body(uxyz_ref, kxyz_ref, xf_ref, g1_ref, w1b_ref, b1_ref,
              y_ref, s_ref):
    b = pl.program_id(0)
    i = pl.program_id(1)
    nt = y_ref.shape[2]
    d = _sqdist(kxyz_ref[0], uxyz_ref[0], N1, nt)
    wt = _top3_weights(d, N1)
    y = jnp.dot(g1_ref[0], wt, preferred_element_type=jnp.float32)
    y = y + jnp.dot(w1b_ref[...], xf_ref[0],
                    preferred_element_type=jnp.float32)
    y = y + b1_ref[...]
    y_ref[0] = y
    _accum_sums(s_ref, y, jnp.logical_and(b == 0, i == 0))


def _col(v):
    return v.reshape(-1, 1)


def kernel(input_xyz, sa1_xyz, sa2_xyz, input_features, sa1_features,
           backbone_feat, fp1_w1, fp1_b1, fp1_g1, fp1_be1, fp1_w2, fp1_b2,
           fp1_g2, fp1_be2, fp2_w1, fp2_b1, fp2_g1, fp2_be1, fp2_w2, fp2_b2,
           fp2_g2, fp2_be2, cls_g, cls_be, cls_w, cls_b):
    f32 = jnp.float32
    sa1_xyz_c = sa1_xyz.transpose(0, 2, 1)
    input_xyz_c = input_xyz.transpose(0, 2, 1)
    w1a_fp1 = fp1_w1[:, :256]
    w1b_fp1 = fp1_w1[:, 256:]
    w1a_fp2 = fp2_w1[:, :256]
    w1b_fp2 = fp2_w1[:, 256:]

    full = lambda shp: pl.BlockSpec(shp, lambda b: tuple(0 for _ in shp))
    perb = lambda shp: pl.BlockSpec(
        (1,) + shp, lambda b: (b,) + tuple(0 for _ in shp))

    M1 = float(B * N1)
    M2 = float(B * N)

    y1, s1 = pl.pallas_call(
        _fp1_body,
        grid=(B,),
        in_specs=[perb((3, N1)), perb((N2, 3)), full((256, 256)),
                  perb((256, N2)), perb((128, N1)), full((256, 128)),
                  full((256, 1))],
        out_specs=[perb((256, N1)), full((256, 2))],
        out_shape=[jax.ShapeDtypeStruct((B, 256, N1), f32),
                   jax.ShapeDtypeStruct((256, 2), f32)],
    )(sa1_xyz_c, sa2_xyz, w1a_fp1, backbone_feat, sa1_features, w1b_fp1,
      _col(fp1_b1))

    z1, s2 = pl.pallas_call(
        functools.partial(_bn_relu_conv_body, M1),
        grid=(B,),
        in_specs=[perb((256, N1)), full((256, 2)), full((256, 1)),
                  full((256, 1)), full((256, 256)), full((256, 1))],
        out_specs=[perb((256, N1)), full((256, 2))],
        out_shape=[jax.ShapeDtypeStruct((B, 256, N1), f32),
                   jax.ShapeDtypeStruct((256, 2), f32)],
    )(y1, s1, _col(fp1_g1), _col(fp1_be1), fp1_w2, _col(fp1_b2))

    g1f = pl.pallas_call(
        functools.partial(_bn_relu_conv_body_nobias, M1),
        grid=(B,),
        in_specs=[perb((256, N1)), full((256, 2)), full((256, 1)),
                  full((256, 1)), full((256, 256))],
        out_specs=perb((256, N1)),
        out_shape=jax.ShapeDtypeStruct((B, 256, N1), f32),
    )(z1, s2, _col(fp1_g2), _col(fp1_be2), w1a_fp2)

    NT = 1024
    nsteps = N // NT
    y2, s3 = pl.pallas_call(
        _fp2_body,
        grid=(B, nsteps),
        in_specs=[
            pl.BlockSpec((1, 3, NT), lambda b, i: (b, 0, i)),
            pl.BlockSpec((1, N1, 3), lambda b, i: (b, 0, 0)),
            pl.BlockSpec((1, DIN, NT), lambda b, i: (b, 0, i)),
            pl.BlockSpec((1, 256, N1), lambda b, i: (b, 0, 0)),
            pl.BlockSpec((256, DIN), lambda b, i: (0, 0)),
            pl.BlockSpec((256, 1), lambda b, i: (0, 0)),
        ],
        out_specs=[pl.BlockSpec((1, 256, NT), lambda b, i: (b, 0, i)),
                   pl.BlockSpec((256, 2), lambda b, i: (0, 0))],
        out_shape=[jax.ShapeDtypeStruct((B, 256, N), f32),
                   jax.ShapeDtypeStruct((256, 2), f32)],
    )(input_xyz_c, sa1_xyz, input_features, g1f, w1b_fp2, _col(fp2_b1))

    z2, s4 = pl.pallas_call(
        functools.partial(_bn_relu_conv_body, M2),
        grid=(B,),
        in_specs=[perb((256, N)), full((256, 2)), full((256, 1)),
                  full((256, 1)), full((256, 256)), full((256, 1))],
        out_specs=[perb((256, N)), full((256, 2))],
        out_shape=[jax.ShapeDtypeStruct((B, 256, N), f32),
                   jax.ShapeDtypeStruct((256, 2), f32)],
    )(y2, s3, _col(fp2_g1), _col(fp2_be1), fp2_w2, _col(fp2_b2))

    f2, s5 = pl.pallas_call(
        functools.partial(_bn_relu_body, M2),
        grid=(B,),
        in_specs=[perb((256, N)), full((256, 2)), full((256, 1)),
                  full((256, 1))],
        out_specs=[perb((256, N)), full((256, 2))],
        out_shape=[jax.ShapeDtypeStruct((B, 256, N), f32),
                   jax.ShapeDtypeStruct((256, 2), f32)],
    )(z2, s4, _col(fp2_g2), _col(fp2_be2))

    pred = pl.pallas_call(
        functools.partial(_bn_relu_conv_body_nosum, M2),
        grid=(B,),
        in_specs=[perb((256, N)), full((256, 2)), full((256, 1)),
                  full((256, 1)), full((NC, 256)), full((NC, 1))],
        out_specs=perb((NC, N)),
        out_shape=jax.ShapeDtypeStruct((B, NC, N), f32),
    )(f2, s5, _col(cls_g), _col(cls_be), cls_w, _col(cls_b))

    return (f2, pred)
